# baseline (speedup 1.0000x reference)
import functools

import jax
import jax.numpy as jnp
from jax import lax
from jax.experimental import pallas as pl
from jax.experimental.pallas import tpu as pltpu
from jax.experimental.pallas import tpu_sc as plsc

_B, _S = 4, 2048
_H = 768
_VOCAB = 260
_HASH_V = 100000
_BASE = 257
_NGRAMS = (3, 4, 5, 6)
_EPS = 1e-12

_NT = _B * _S
_NC = 2
_NSC = 16
_NW = _NC * _NSC
_K = 16
_HALO = 16

_NSPLIT = 1
_HTOK = _NT // _NSPLIT

_BT = 2048
_NB = _HTOK // _BT


def _make_sc_body(h_off, ntok):
    tpw = ntok // _NW
    nchunk = tpw // _K

    def _sc_body(ids_hbm, t3, t4, t5, t6, comb_hbm,
                 ids_v, hv_v, a0, a1, a2, a3, b0, b1, b2, b3,
                 sga, sgb, soa, sob):
        wid = lax.axis_index("s") * _NC + lax.axis_index("c")
        lt0 = wid * tpw
        t0 = lt0 + h_off
        pltpu.sync_copy(ids_hbm.at[pl.ds(t0, tpw)],
                        ids_v.at[pl.ds(_HALO, tpw)])

        @pl.when(t0 > 0)
        def _load_halo():
            hbase = pl.multiple_of(jnp.maximum(t0 - _HALO, 0), _HALO)
            pltpu.sync_copy(ids_hbm.at[pl.ds(hbase, _HALO)],
                            ids_v.at[pl.ds(0, _HALO)])

        p0 = lax.rem(t0, _S)
        iota = lax.iota(jnp.int32, 16)

        def hash_group(g, carry):
            shs = []
            for j in range(max(_NGRAMS)):
                shs.append(ids_v[pl.ds(_HALO + g * 16 - j, 16)])
            posv = iota + (p0 + g * 16)
            for ni, n in enumerate(_NGRAMS):
                tot = shs[0] * pow(_BASE, n - 1, _HASH_V)
                for j in range(1, n):
                    tot = tot + shs[j] * pow(_BASE, n - 1 - j, _HASH_V)
                hv = jnp.where(posv < (n - 1), shs[0], lax.rem(tot, _HASH_V))
                hv_v[ni, pl.ds(g * 16, 16)] = hv
            return carry

        ngroup_01 = 2 * (_K // 16)
        lax.fori_loop(0, ngroup_01, hash_group, 0)

        tables = (t3, t4, t5, t6)
        seta = (a0, a1, a2, a3)
        setb = (b0, b1, b2, b3)

        def issue(c, bufs, sem):
            for ti in range(4):
                pltpu.async_copy(
                    tables[ti].at[hv_v.at[ti, pl.ds(c * _K, _K)]],
                    bufs[ti], sem)

        def drain(c, bufs, sem):
            for ti in range(4):
                pltpu.make_async_copy(
                    tables[ti].at[hv_v.at[ti, pl.ds(c * _K, _K)]],
                    bufs[ti], sem).wait()

        def out_wait(bufs, osem):
            pltpu.make_async_copy(bufs[0], comb_hbm.at[pl.ds(0, _K)],
                                  osem).wait()

        def accum_and_put(c, bufs, osem):
            def acc_row(r, carry2):
                for q in range(_H // 16):
                    sl = pl.ds(q * 16, 16)
                    plsc.addupdate(bufs[0].at[r, sl],
                                   (bufs[1][r, sl] + bufs[2][r, sl])
                                   + bufs[3][r, sl])
                return carry2

            lax.fori_loop(0, _K, acc_row, 0)
            pltpu.async_copy(bufs[0], comb_hbm.at[pl.ds(lt0 + c * _K, _K)],
                             osem)

        issue(0, seta, sga)

        def body(i, carry):
            ca = 2 * i
            issue(ca + 1, setb, sgb)

            @pl.when(i == 0)
            def _hash_rest():
                lax.fori_loop(ngroup_01, tpw // 16, hash_group, 0)
            drain(ca, seta, sga)

            @pl.when(i > 0)
            def _wa():
                out_wait(seta, soa)

            accum_and_put(ca, seta, soa)

            @pl.when(i < nchunk // 2 - 1)
            def _ia():
                issue(ca + 2, seta, sga)

            drain(ca + 1, setb, sgb)

            @pl.when(i > 0)
            def _wb():
                out_wait(setb, sob)

            accum_and_put(ca + 1, setb, sob)
            return carry

        lax.fori_loop(0, nchunk // 2, body, 0)
        out_wait(seta, soa)
        out_wait(setb, sob)

    return _sc_body


def _sc_combine(h_off, ntok, ids_flat, t3, t4, t5, t6):
    tpw = ntok // _NW
    run = functools.partial(
        pl.kernel,
        mesh=plsc.VectorSubcoreMesh(core_axis_name="c", subcore_axis_name="s"),
        out_type=jax.ShapeDtypeStruct((ntok, _H), jnp.float32),
        scratch_types=(
            [pltpu.VMEM((tpw + _HALO,), jnp.int32),
             pltpu.VMEM((len(_NGRAMS), tpw), jnp.int32)]
            + [pltpu.VMEM((_K, _H), jnp.float32) for _ in range(8)]
            + [pltpu.SemaphoreType.DMA for _ in range(4)]
        ),
    )(_make_sc_body(h_off, ntok))
    return run(ids_flat, t3, t4, t5, t6)


def _tc_body(ids_ref, comb_ref, pos_ref, w_ref, g_ref, be_ref, o_ref):
    ids = ids_ref[0, 0, :]
    onehot = (ids[:, None]
              == lax.broadcasted_iota(jnp.int32, (_BT, _VOCAB), 1)
              ).astype(jnp.bfloat16)
    word = jnp.dot(onehot, w_ref[...], preferred_element_type=jnp.float32)
    pbase = lax.rem(pl.program_id(0), _S // _BT) * _BT
    pos = pos_ref[pl.ds(pbase, _BT), :]
    emb = word + pos + comb_ref[...] * (1.0 / len(_NGRAMS))
    mu = jnp.mean(emb, axis=1, keepdims=True)
    xc = emb - mu
    var = jnp.mean(xc * xc, axis=1, keepdims=True)
    inv = lax.rsqrt(var + _EPS)
    o_ref[...] = xc * inv * g_ref[...] + be_ref[...]


def _tc_finish(ids_slice, comb, pos_emb, word_emb, gamma2, beta2):
    return pl.pallas_call(
        _tc_body,
        grid=(_NB,),
        in_specs=[
            pl.BlockSpec((1, 1, _BT), lambda i: (i, 0, 0)),
            pl.BlockSpec((_BT, _H), lambda i: (i, 0)),
            pl.BlockSpec((_S, _H), lambda i: (0, 0)),
            pl.BlockSpec((_VOCAB, _H), lambda i: (0, 0)),
            pl.BlockSpec((1, _H), lambda i: (0, 0)),
            pl.BlockSpec((1, _H), lambda i: (0, 0)),
        ],
        out_specs=pl.BlockSpec((_BT, _H), lambda i: (i, 0)),
        out_shape=jax.ShapeDtypeStruct((_HTOK, _H), jnp.float32),
    )(ids_slice.reshape(_NB, 1, _BT), comb, pos_emb,
      word_emb.astype(jnp.bfloat16), gamma2, beta2)


def kernel(input_ids, word_emb, pos_emb, hash_emb_3, hash_emb_4, hash_emb_5,
           hash_emb_6, gamma, beta):
    ids_flat = input_ids.reshape(_NT)
    gamma2 = gamma.reshape(1, _H)
    beta2 = beta.reshape(1, _H)
    outs = []
    for h in range(_NSPLIT):
        comb = _sc_combine(h * _HTOK, _HTOK, ids_flat, hash_emb_3,
                           hash_emb_4, hash_emb_5, hash_emb_6)
        ids_slice = lax.slice(ids_flat, (h * _HTOK,), ((h + 1) * _HTOK,))
        outs.append(_tc_finish(ids_slice, comb, pos_emb, word_emb,
                               gamma2, beta2))
    return jnp.concatenate(outs, axis=0).reshape(_B, _S, _H)
